# baseline (speedup 1.0000x reference)
import functools

import jax
import jax.numpy as jnp
from jax import lax
from jax.experimental import pallas as pl
from jax.experimental.pallas import tpu as pltpu
from jax.experimental.pallas import tpu_sc as plsc

SEQ = 4096
HID = 768
NSPANS = 2048
BLK = 128
NBLK = SEQ // BLK
SPAD = SEQ + BLK
OPAD = NBLK + 8

NC = 2
NS = 16
NW = NC * NS
SP_PER_W = NSPANS // NW
GRP = 16
NGRP = SP_PER_W // GRP
NBUF = 2
NCH = HID // 16


NCHK = 4
CHR = SEQ // NCHK
BPC = NBLK // NCHK


def _prefix_body(emb_hbm, w_ref, b_ref, lix_hbm, lie_ref, offe_ref,
                 emb_v, stg0, stg1, zbuf,
                 is0, is1, is2, is3, os0, os1, zs):
    isem = [is0, is1, is2, is3]
    osem = [os0, os1]
    stg = [stg0, stg1]

    in_copies = [
        pltpu.async_copy(emb_hbm.at[pl.ds(c * CHR, CHR), :],
                         emb_v.at[pl.ds(c * CHR, CHR), :], isem[c])
        for c in range(NCHK)
    ]

    w = w_ref[...]
    scores = []
    gmax = None
    for c in range(NCHK):
        in_copies[c].wait()
        blkc = emb_v[pl.ds(c * CHR, CHR), :]
        sc = jnp.sum(blkc * w, axis=1, keepdims=True) + b_ref[0, 0]
        scores.append(sc)
        m = jnp.max(sc)
        gmax = m if gmax is None else jnp.maximum(gmax, m)

    row = lax.broadcasted_iota(jnp.int32, (BLK, BLK), 0)
    col = lax.broadcasted_iota(jnp.int32, (BLK, BLK), 1)
    tinc = (row >= col).astype(jnp.float32)
    rowb = lax.broadcasted_iota(jnp.int32, (NBLK, NBLK), 0)
    colb = lax.broadcasted_iota(jnp.int32, (NBLK, NBLK), 1)
    texc = (rowb > colb).astype(jnp.float32)

    zbuf[...] = jnp.zeros((BLK, HID), jnp.float32)
    zcopy = pltpu.async_copy(zbuf, lix_hbm.at[pl.ds(SEQ, BLK), :], zs)

    carry = jnp.zeros((1, HID), jnp.float32)
    se_rows = []
    out_handles = {}
    for c in range(NCHK):
        if c >= 2:
            out_handles.pop(c - 2).wait()
        buf = stg[c % 2]
        for j in range(BPC):
            k = c * BPC + j
            ek = jnp.exp(scores[c][j * BLK:(j + 1) * BLK] - gmax)
            xk = emb_v[pl.ds(k * BLK, BLK), :] * ek
            lixk = jnp.dot(tinc, xk, preferred_element_type=jnp.float32)
            liek = jnp.dot(tinc, ek, preferred_element_type=jnp.float32,
                           precision=lax.Precision.HIGHEST)
            lie_ref[k:k + 1, :] = jnp.transpose(liek)
            se_rows.append(liek[BLK - 1:BLK, :])
            buf[j * BLK:(j + 1) * BLK, :] = lixk + carry
            carry = carry + lixk[BLK - 1:BLK, :]
        out_handles[c] = pltpu.async_copy(
            buf, lix_hbm.at[pl.ds(c * CHR, CHR), :], osem[c % 2])

    lie_ref[NBLK:OPAD, :] = jnp.zeros((OPAD - NBLK, BLK), jnp.float32)
    se = jnp.concatenate(se_rows, axis=0)
    offe_cols = jnp.dot(texc, se, preferred_element_type=jnp.float32,
                        precision=lax.Precision.HIGHEST)
    offe_ref[...] = jnp.zeros((8, BLK), jnp.float32)
    offe_ref[0:1, 0:NBLK] = jnp.transpose(offe_cols)

    zcopy.wait()
    for c in list(out_handles):
        out_handles.pop(c).wait()


def _prefix_stage(emb, w, b):
    return pl.pallas_call(
        _prefix_body,
        in_specs=[
            pl.BlockSpec(memory_space=pl.ANY),
            pl.BlockSpec(memory_space=pltpu.MemorySpace.VMEM),
            pl.BlockSpec(memory_space=pltpu.MemorySpace.VMEM),
        ],
        out_specs=(
            pl.BlockSpec(memory_space=pl.ANY),
            pl.BlockSpec(memory_space=pltpu.MemorySpace.VMEM),
            pl.BlockSpec(memory_space=pltpu.MemorySpace.VMEM),
        ),
        out_shape=(
            jax.ShapeDtypeStruct((SPAD, HID), jnp.float32),
            jax.ShapeDtypeStruct((OPAD, BLK), jnp.float32),
            jax.ShapeDtypeStruct((8, BLK), jnp.float32),
        ),
        scratch_shapes=[
            pltpu.VMEM((SEQ, HID), jnp.float32),
            pltpu.VMEM((CHR, HID), jnp.float32),
            pltpu.VMEM((CHR, HID), jnp.float32),
            pltpu.VMEM((BLK, HID), jnp.float32),
        ] + [pltpu.SemaphoreType.DMA] * 7,
    )(emb, w, b)


def _span_body(lix_hbm, lie_hbm, offe_hbm, emb_hbm, spans_hbm,
               out_hbm, spans_v, lie_v, offe_v,
               big0, big1, gs0, gs1,
               ei0, si0, pi0, ei1, si1, pi1,
               gsem0, gsem1, osem0, osem1):
    big = [big0, big1]
    gs = [gs0, gs1]
    eidx = [ei0, ei1]
    sidx = [si0, si1]
    spidx = [pi0, pi1]
    gsem = [gsem0, gsem1]
    osem = [osem0, osem1]

    wid = lax.axis_index("s") * NC + lax.axis_index("c")
    pltpu.sync_copy(spans_hbm.at[pl.ds(wid * SP_PER_W, SP_PER_W), :], spans_v)
    pltpu.sync_copy(lie_hbm, lie_v)
    pltpu.sync_copy(offe_hbm, offe_v)

    lane = lax.iota(jnp.int32, 16)
    zz = jnp.zeros((16,), jnp.int32)
    zo = jnp.ones((16,), jnp.int32)

    def span_vecs(t):
        gidx = jnp.minimum(t * GRP + lane, SP_PER_W - 1)
        s_vec = plsc.load_gather(spans_v, [gidx, zz])
        e_vec = plsc.load_gather(spans_v, [gidx, zo])
        s_is0 = s_vec == 0
        sp_vec = jnp.where(s_is0, SEQ, s_vec - 1)
        be_vec = lax.shift_right_logical(e_vec, 7)
        bsp_vec = jnp.where(s_is0, NBLK,
                            lax.shift_right_logical(s_vec - 1, 7))
        return s_vec, e_vec, sp_vec, be_vec, bsp_vec

    def issue_gathers(t, k):
        s_vec, e_vec, sp_vec, _, _ = span_vecs(t)
        eidx[k][...] = e_vec
        sidx[k][...] = s_vec
        spidx[k][...] = sp_vec
        ei = eidx[k].at[pl.ds(0, GRP)]
        si = sidx[k].at[pl.ds(0, GRP)]
        pi = spidx[k].at[pl.ds(0, GRP)]
        return [
            pltpu.async_copy(emb_hbm.at[si], big[k].at[:, pl.ds(0, HID)],
                             gsem[k]),
            pltpu.async_copy(emb_hbm.at[ei], big[k].at[:, pl.ds(HID, HID)],
                             gsem[k]),
            pltpu.async_copy(lix_hbm.at[ei], big[k].at[:, pl.ds(2 * HID, HID)],
                             gsem[k]),
            pltpu.async_copy(lix_hbm.at[pi], gs[k], gsem[k]),
        ]

    pend_g = {0: issue_gathers(0, 0)}
    pend_o = {}

    for g in range(NGRP):
        k = g % NBUF
        kn = (g + 1) % NBUF
        if g + 1 < NGRP:
            for c in pend_o.pop(kn, ()):
                c.wait()
            pend_g[kn] = issue_gathers(g + 1, kn)
        for c in pend_g.pop(k):
            c.wait()

        _, e_vec, sp_vec, be_vec, bsp_vec = span_vecs(g)
        c127 = jnp.full((16,), 127, jnp.int32)
        den = (plsc.load_gather(offe_v, [zz, be_vec])
               + plsc.load_gather(lie_v, [lax.shift_right_logical(e_vec, 7),
                                          e_vec & c127])
               - plsc.load_gather(offe_v, [zz, bsp_vec])
               - plsc.load_gather(lie_v, [lax.shift_right_logical(sp_vec, 7),
                                          sp_vec & c127]))
        inv_vec = 1.0 / den
        inv = [inv_vec[j] for j in range(GRP)]

        bigk, gsk = big[k], gs[k]

        def chunk(c, carry):
            for u in range(2):
                o = c * 32 + u * 16
                for j in range(GRP):
                    num = (bigk[j, pl.ds(2 * HID + o, 16)]
                           - gsk[j, pl.ds(o, 16)])
                    bigk[j, pl.ds(2 * HID + o, 16)] = num * inv[j]
            return carry

        lax.fori_loop(0, NCH // 2, chunk, 0)

        base = wid * SP_PER_W + g * GRP
        pend_o[k] = [
            pltpu.async_copy(big[k], out_hbm.at[pl.ds(base, GRP), :],
                             osem[k]),
        ]

    for k in list(pend_o):
        for c in pend_o.pop(k):
            c.wait()


@functools.cache
def _make_span_stage():
    row_bufs = ([pltpu.VMEM((GRP, 3 * HID), jnp.float32)] * NBUF
                + [pltpu.VMEM((GRP, HID), jnp.float32)] * NBUF)
    idx_bufs = [pltpu.VMEM((16,), jnp.int32)] * (3 * NBUF)
    sems = [pltpu.SemaphoreType.DMA] * (2 * NBUF)
    return functools.partial(
        pl.kernel,
        out_type=jax.ShapeDtypeStruct((NSPANS, 3 * HID), jnp.float32),
        mesh=plsc.VectorSubcoreMesh(core_axis_name="c", subcore_axis_name="s"),
        compiler_params=pltpu.CompilerParams(needs_layout_passes=False),
        scratch_types=[
            pltpu.VMEM((SP_PER_W, 2), jnp.int32),
            pltpu.VMEM((OPAD, BLK), jnp.float32),
            pltpu.VMEM((8, BLK), jnp.float32),
        ] + row_bufs + idx_bufs + sems,
    )(_span_body)


@jax.jit
def kernel(embeddings, all_spans, W, b):
    emb = embeddings[0]
    w2 = W.reshape(1, HID)
    b2 = b.reshape(1, 1)
    lix, lie, offe = _prefix_stage(emb, w2, b2)
    return _make_span_stage()(lix, lie, offe, emb,
                              all_spans.astype(jnp.int32))


# speedup vs baseline: 1.0861x; 1.0861x over previous
import functools

import jax
import jax.numpy as jnp
from jax import lax
from jax.experimental import pallas as pl
from jax.experimental.pallas import tpu as pltpu
from jax.experimental.pallas import tpu_sc as plsc

SEQ = 4096
HID = 768
NSPANS = 2048
BLK = 128
NBLK = SEQ // BLK
SPAD = SEQ + BLK
OPAD = NBLK + 8

NC = 2
NS = 16
NW = NC * NS
SP_PER_W = NSPANS // NW
GRP = 8
NGRP = SP_PER_W // GRP
NBUF = 3
NCH = HID // 16


NCHK = 4
CHR = SEQ // NCHK
BPC = NBLK // NCHK


def _prefix_body(emb_hbm, w_ref, b_ref, lix_hbm, lie_ref, offe_ref,
                 emb_v, stg0, stg1, zbuf,
                 is0, is1, is2, is3, os0, os1, zs):
    isem = [is0, is1, is2, is3]
    osem = [os0, os1]
    stg = [stg0, stg1]

    in_copies = [
        pltpu.async_copy(emb_hbm.at[pl.ds(c * CHR, CHR), :],
                         emb_v.at[pl.ds(c * CHR, CHR), :], isem[c])
        for c in range(NCHK)
    ]

    w = w_ref[...]
    scores = []
    gmax = None
    for c in range(NCHK):
        in_copies[c].wait()
        blkc = emb_v[pl.ds(c * CHR, CHR), :]
        sc = jnp.sum(blkc * w, axis=1, keepdims=True) + b_ref[0, 0]
        scores.append(sc)
        m = jnp.max(sc)
        gmax = m if gmax is None else jnp.maximum(gmax, m)

    row = lax.broadcasted_iota(jnp.int32, (BLK, BLK), 0)
    col = lax.broadcasted_iota(jnp.int32, (BLK, BLK), 1)
    tinc = (row >= col).astype(jnp.float32)
    rowb = lax.broadcasted_iota(jnp.int32, (NBLK, NBLK), 0)
    colb = lax.broadcasted_iota(jnp.int32, (NBLK, NBLK), 1)
    texc = (rowb > colb).astype(jnp.float32)

    zbuf[...] = jnp.zeros((BLK, HID), jnp.float32)
    zcopy = pltpu.async_copy(zbuf, lix_hbm.at[pl.ds(SEQ, BLK), :], zs)

    carry = jnp.zeros((1, HID), jnp.float32)
    se_rows = []
    out_handles = {}
    for c in range(NCHK):
        if c >= 2:
            out_handles.pop(c - 2).wait()
        buf = stg[c % 2]
        for j in range(BPC):
            k = c * BPC + j
            ek = jnp.exp(scores[c][j * BLK:(j + 1) * BLK] - gmax)
            xk = emb_v[pl.ds(k * BLK, BLK), :] * ek
            lixk = jnp.dot(tinc, xk, preferred_element_type=jnp.float32)
            liek = jnp.dot(tinc, ek, preferred_element_type=jnp.float32,
                           precision=lax.Precision.HIGHEST)
            lie_ref[k:k + 1, :] = jnp.transpose(liek)
            se_rows.append(liek[BLK - 1:BLK, :])
            buf[j * BLK:(j + 1) * BLK, :] = lixk + carry
            carry = carry + lixk[BLK - 1:BLK, :]
        out_handles[c] = pltpu.async_copy(
            buf, lix_hbm.at[pl.ds(c * CHR, CHR), :], osem[c % 2])

    lie_ref[NBLK:OPAD, :] = jnp.zeros((OPAD - NBLK, BLK), jnp.float32)
    se = jnp.concatenate(se_rows, axis=0)
    offe_cols = jnp.dot(texc, se, preferred_element_type=jnp.float32,
                        precision=lax.Precision.HIGHEST)
    offe_ref[...] = jnp.zeros((8, BLK), jnp.float32)
    offe_ref[0:1, 0:NBLK] = jnp.transpose(offe_cols)

    zcopy.wait()
    for c in list(out_handles):
        out_handles.pop(c).wait()


def _prefix_stage(emb, w, b):
    return pl.pallas_call(
        _prefix_body,
        in_specs=[
            pl.BlockSpec(memory_space=pl.ANY),
            pl.BlockSpec(memory_space=pltpu.MemorySpace.VMEM),
            pl.BlockSpec(memory_space=pltpu.MemorySpace.VMEM),
        ],
        out_specs=(
            pl.BlockSpec(memory_space=pl.ANY),
            pl.BlockSpec(memory_space=pltpu.MemorySpace.VMEM),
            pl.BlockSpec(memory_space=pltpu.MemorySpace.VMEM),
        ),
        out_shape=(
            jax.ShapeDtypeStruct((SPAD, HID), jnp.float32),
            jax.ShapeDtypeStruct((OPAD, BLK), jnp.float32),
            jax.ShapeDtypeStruct((8, BLK), jnp.float32),
        ),
        scratch_shapes=[
            pltpu.VMEM((SEQ, HID), jnp.float32),
            pltpu.VMEM((CHR, HID), jnp.float32),
            pltpu.VMEM((CHR, HID), jnp.float32),
            pltpu.VMEM((BLK, HID), jnp.float32),
        ] + [pltpu.SemaphoreType.DMA] * 7,
    )(emb, w, b)


def _span_body(lix_hbm, lie_hbm, offe_hbm, emb_hbm, spans_hbm,
               out_hbm, spans_v, lie_v, offe_v,
               big0, big1, big2, gs0, gs1, gs2,
               ei0, si0, pi0, ei1, si1, pi1, ei2, si2, pi2,
               gsem0, gsem1, gsem2, osem0, osem1, osem2):
    big = [big0, big1, big2]
    gs = [gs0, gs1, gs2]
    eidx = [ei0, ei1, ei2]
    sidx = [si0, si1, si2]
    spidx = [pi0, pi1, pi2]
    gsem = [gsem0, gsem1, gsem2]
    osem = [osem0, osem1, osem2]

    wid = lax.axis_index("s") * NC + lax.axis_index("c")
    pltpu.sync_copy(spans_hbm.at[pl.ds(wid * SP_PER_W, SP_PER_W), :], spans_v)
    pltpu.sync_copy(lie_hbm, lie_v)
    pltpu.sync_copy(offe_hbm, offe_v)

    lane = lax.iota(jnp.int32, 16)
    zz = jnp.zeros((16,), jnp.int32)
    zo = jnp.ones((16,), jnp.int32)

    def span_vecs(t):
        gidx = jnp.minimum(t * GRP + lane, SP_PER_W - 1)
        s_vec = plsc.load_gather(spans_v, [gidx, zz])
        e_vec = plsc.load_gather(spans_v, [gidx, zo])
        s_is0 = s_vec == 0
        sp_vec = jnp.where(s_is0, SEQ, s_vec - 1)
        be_vec = lax.shift_right_logical(e_vec, 7)
        bsp_vec = jnp.where(s_is0, NBLK,
                            lax.shift_right_logical(s_vec - 1, 7))
        return s_vec, e_vec, sp_vec, be_vec, bsp_vec

    def issue_gathers(t, k):
        s_vec, e_vec, sp_vec, _, _ = span_vecs(t)
        eidx[k][...] = e_vec
        sidx[k][...] = s_vec
        spidx[k][...] = sp_vec
        ei = eidx[k].at[pl.ds(0, GRP)]
        si = sidx[k].at[pl.ds(0, GRP)]
        pi = spidx[k].at[pl.ds(0, GRP)]
        return [
            pltpu.async_copy(emb_hbm.at[si], big[k].at[:, pl.ds(0, HID)],
                             gsem[k]),
            pltpu.async_copy(emb_hbm.at[ei], big[k].at[:, pl.ds(HID, HID)],
                             gsem[k]),
            pltpu.async_copy(lix_hbm.at[ei], big[k].at[:, pl.ds(2 * HID, HID)],
                             gsem[k]),
            pltpu.async_copy(lix_hbm.at[pi], gs[k], gsem[k]),
        ]

    pend_g = {0: issue_gathers(0, 0)}
    pend_o = {}

    for g in range(NGRP):
        k = g % NBUF
        kn = (g + 1) % NBUF
        if g + 1 < NGRP:
            for c in pend_o.pop(kn, ()):
                c.wait()
            pend_g[kn] = issue_gathers(g + 1, kn)
        for c in pend_g.pop(k):
            c.wait()

        _, e_vec, sp_vec, be_vec, bsp_vec = span_vecs(g)
        c127 = jnp.full((16,), 127, jnp.int32)
        den = (plsc.load_gather(offe_v, [zz, be_vec])
               + plsc.load_gather(lie_v, [lax.shift_right_logical(e_vec, 7),
                                          e_vec & c127])
               - plsc.load_gather(offe_v, [zz, bsp_vec])
               - plsc.load_gather(lie_v, [lax.shift_right_logical(sp_vec, 7),
                                          sp_vec & c127]))
        inv_vec = 1.0 / den
        inv = [inv_vec[j] for j in range(GRP)]

        bigk, gsk = big[k], gs[k]

        def chunk(c, carry):
            for u in range(2):
                o = c * 32 + u * 16
                for j in range(GRP):
                    num = (bigk[j, pl.ds(2 * HID + o, 16)]
                           - gsk[j, pl.ds(o, 16)])
                    bigk[j, pl.ds(2 * HID + o, 16)] = num * inv[j]
            return carry

        lax.fori_loop(0, NCH // 2, chunk, 0)

        base = wid * SP_PER_W + g * GRP
        pend_o[k] = [
            pltpu.async_copy(big[k], out_hbm.at[pl.ds(base, GRP), :],
                             osem[k]),
        ]

    for k in list(pend_o):
        for c in pend_o.pop(k):
            c.wait()


@functools.cache
def _make_span_stage():
    row_bufs = ([pltpu.VMEM((GRP, 3 * HID), jnp.float32)] * NBUF
                + [pltpu.VMEM((GRP, HID), jnp.float32)] * NBUF)
    idx_bufs = [pltpu.VMEM((16,), jnp.int32)] * (3 * NBUF)
    sems = [pltpu.SemaphoreType.DMA] * (2 * NBUF)
    return functools.partial(
        pl.kernel,
        out_type=jax.ShapeDtypeStruct((NSPANS, 3 * HID), jnp.float32),
        mesh=plsc.VectorSubcoreMesh(core_axis_name="c", subcore_axis_name="s"),
        compiler_params=pltpu.CompilerParams(needs_layout_passes=False),
        scratch_types=[
            pltpu.VMEM((SP_PER_W, 2), jnp.int32),
            pltpu.VMEM((OPAD, BLK), jnp.float32),
            pltpu.VMEM((8, BLK), jnp.float32),
        ] + row_bufs + idx_bufs + sems,
    )(_span_body)


@jax.jit
def kernel(embeddings, all_spans, W, b):
    emb = embeddings[0]
    w2 = W.reshape(1, HID)
    b2 = b.reshape(1, 1)
    lix, lie, offe = _prefix_stage(emb, w2, b2)
    return _make_span_stage()(lix, lie, offe, emb,
                              all_spans.astype(jnp.int32))
